# baseline (speedup 1.0000x reference)
import functools

import jax
import jax.numpy as jnp
from jax import lax
from jax.experimental import pallas as pl
from jax.experimental.pallas import tpu as pltpu
from jax.experimental.pallas import tpu_sc as plsc

EMB = 16
B = 16384
NU = 1000000
NM = 100000
NC, NS = 2, 16
NW = NC * NS
L = 16
ROWS = B // 128
CHUNK = 128
CPW = 4

BLKW = 1536
TBASE_U = (NU // 128) * 128
TBASE_M = (NM // 128) * 128
RPW_U = 245 * 128
RPW_M = 25 * 128
NBLK_U = 22
NBLK_M = 4
OFFCAP_U = ((NU - BLKW) // 128) * 128
OFFCAP_M = ((NM - BLKW) // 128) * 128
CAP = B
SLOTC = CAP // 128

_mesh = plsc.VectorSubcoreMesh(core_axis_name="c", subcore_axis_name="s")


def _i32(x):
    return jnp.asarray(x, jnp.int32)


@functools.partial(
    pl.kernel,
    mesh=_mesh,
    out_type=(
        jax.ShapeDtypeStruct((2, NW * SLOTC, 8, 128), jnp.float32),
        jax.ShapeDtypeStruct((2, NW * SLOTC, 8, 128), jnp.float32),
        jax.ShapeDtypeStruct((NW, SLOTC, 128), jnp.int32),
        jax.ShapeDtypeStruct((NW, SLOTC, 128), jnp.int32),
        jax.ShapeDtypeStruct((NW, 128), jnp.int32),
        jax.ShapeDtypeStruct((NW, 128), jnp.int32),
    ),
    scratch_types=(
        pltpu.VMEM((B,), jnp.int32),
        pltpu.VMEM((B + 16,), jnp.int32),
        pltpu.VMEM((48,), jnp.int32),
        pltpu.VMEM((48,), jnp.int32),
        pltpu.VMEM((L, BLKW + 128), jnp.float32),
        pltpu.VMEM((L, BLKW + 128), jnp.float32),
        pltpu.VMEM((2, 16, 8, 128), jnp.float32),
        pltpu.VMEM((2, 8, 128), jnp.int32),
        pltpu.VMEM((128,), jnp.int32),
        pltpu.SemaphoreType.DMA,
        pltpu.SemaphoreType.DMA,
    ),
    compiler_params=pltpu.CompilerParams(use_tc_tiling_on_sc=True,
                                         needs_layout_passes=False),
)
def _sc_k1(uids, mids, uembT, membT, tailu, tailm,
           ucomp, mcomp, upos_o, mpos_o, ucnt_o, mcnt_o,
           idsb, apos, pend_i, pend_p, blkA, blkB, stage, pstage, cstage,
           semA, semB):
    wid = lax.axis_index("s") * NC + lax.axis_index("c")
    lanes = lax.iota(jnp.int32, L)

    def one_table(ids_hbm, tab, tail, tbase, rpw, nblk, offcap,
                  comp_o, pos_o, cnt_o):
        lo = wid * rpw
        hi = lo + rpw
        pltpu.sync_copy(ids_hbm, idsb)
        pltpu.sync_copy(tail, blkA.at[:, pl.ds(BLKW, 128)])
        pltpu.sync_copy(tail, blkB.at[:, pl.ds(BLKW, 128)])

        def scan_body(g, cnt):
            ids16 = idsb[pl.ds(g * L, L)]
            inr = (ids16 >= lo) & (ids16 < hi)
            c = plsc.cumsum(inr.astype(jnp.int32))
            offs = jnp.full((L,), cnt, jnp.int32) + c - 1
            pos16 = jnp.full((L,), g * L, jnp.int32) + lanes
            plsc.store_scatter(apos, [offs], pos16, mask=inr)
            return cnt + c[L - 1]

        cnt = lax.fori_loop(0, B // L, scan_body, _i32(0), unroll=4)
        nvec = (cnt + (L - 1)) // L

        def ids_of(pos16):
            return plsc.load_gather(idsb, [jnp.clip(pos16, 0, B - 1)])

        def extract_group(buf, base, k, koff, off2, kmask):
            t16p = pend_p[pl.ds(base, L)]
            t16i = pend_i[pl.ds(base, L)]
            lcol = jnp.where(t16i >= tbase,
                             t16i - (tbase - BLKW), t16i - off2)
            lcol = jnp.clip(lcol, 0, BLKW + 127)
            slots = jnp.full((L,), koff, jnp.int32) + lanes
            smask = (lanes < k) if kmask else None
            plsc.store_scatter(
                pstage,
                [(slots >> 10) & 1, (slots >> 7) & 7, slots & 127],
                t16p, mask=smask)
            for t in range(L):
                cs = koff + t
                col = plsc.load_gather(
                    buf, [lanes, jnp.full((L,), lcol[t], jnp.int32)])
                m = None
                if kmask:
                    m = (jnp.full((L,), t, jnp.int32)
                         < jnp.full((L,), k, jnp.int32))
                plsc.store_scatter(
                    stage,
                    [lanes >> 3, jnp.full((L,), (cs >> 7) & 15, jnp.int32),
                     lanes & 7, jnp.full((L,), cs & 127, jnp.int32)],
                    col, mask=m)

        def flush(f):
            c = f & 1
            pltpu.sync_copy(
                stage.at[:, pl.ds(c * 8, 8)],
                comp_o.at[:, pl.ds(wid * SLOTC + f * 8, 8)])
            pltpu.sync_copy(pstage.at[c], pos_o.at[wid, pl.ds(f * 8, 8), :])

        def append(dst, x, m, base):
            c = plsc.cumsum(m.astype(jnp.int32))
            offs = jnp.full((L,), base, jnp.int32) + c - 1
            plsc.store_scatter(dst, [offs], x, mask=m)

        def start(bi, buf, sem):
            off = lo + bi * BLKW
            off2 = jnp.minimum(off, offcap)
            pltpu.async_copy(tab.at[:, pl.ds(off2, BLKW)],
                             buf.at[:, pl.ds(0, BLKW)], sem)

        def drain(buf, sem):
            pltpu.make_async_copy(tab.at[:, pl.ds(0, BLKW)],
                                  buf.at[:, pl.ds(0, BLKW)], sem).wait()

        def process(buf, bi, s):
            off = lo + bi * BLKW
            off2 = jnp.minimum(off, offcap)

            def scan_blk(v, carry):
                s1, p = carry
                pos16 = apos[pl.ds(v * L, L)]
                ids16 = ids_of(pos16)
                valid = (jnp.full((L,), v * L, jnp.int32) + lanes) < cnt
                inb = (ids16 >= off) & (ids16 < off + BLKW) & valid
                kk = plsc.all_reduce_population_count(inb)[0]

                @pl.when(kk > 0)
                def _():
                    append(pend_i, ids16, inb, p)
                    append(pend_p, pos16, inb, p)

                p2 = p + kk

                @pl.when(p2 >= L)
                def _():
                    extract_group(buf, p2 - L, L, s1, off2, kmask=False)

                consumed = jnp.where(p2 >= L, L, 0)
                s2 = s1 + consumed

                @pl.when((s2 >> 10) != (s1 >> 10))
                def _():
                    flush(s1 >> 10)

                return (s2, p2 - consumed)

            s_a, p_a = lax.fori_loop(0, nvec, scan_blk, (s, _i32(0)))

            @pl.when(p_a > 0)
            def _():
                extract_group(buf, 0, p_a, s_a, off2, kmask=True)

            s_b = s_a + p_a

            @pl.when((s_b >> 10) != (s_a >> 10))
            def _():
                flush(s_a >> 10)

            return s_b

        start(_i32(0), blkA, semA)
        start(_i32(1), blkB, semB)

        def pair_iter(bp, s):
            bi0 = bp * 2
            drain(blkA, semA)
            s = process(blkA, bi0, s)
            start(bi0 + 2, blkA, semA)
            drain(blkB, semB)
            s = process(blkB, bi0 + 1, s)
            start(bi0 + 3, blkB, semB)
            return s

        s_fin = lax.fori_loop(0, nblk // 2, pair_iter, _i32(0))
        drain(blkA, semA)
        drain(blkB, semB)

        @pl.when((s_fin & 1023) != 0)
        def _():
            flush(s_fin >> 10)

        for g in range(8):
            cstage[pl.ds(g * L, L)] = jnp.full((L,), cnt, jnp.int32)
        pltpu.sync_copy(cstage, cnt_o.at[wid])

    one_table(uids, uembT, tailu, TBASE_U, RPW_U, NBLK_U, OFFCAP_U,
              ucomp, upos_o, ucnt_o)
    one_table(mids, membT, tailm, TBASE_M, RPW_M, NBLK_M, OFFCAP_M,
              mcomp, mpos_o, mcnt_o)


@functools.partial(
    pl.kernel,
    mesh=_mesh,
    out_type=(
        jax.ShapeDtypeStruct((B + 128, EMB), jnp.float32),
        jax.ShapeDtypeStruct((B + 128, EMB), jnp.float32),
        jax.ShapeDtypeStruct((ROWS, CHUNK), jnp.float32),
        jax.ShapeDtypeStruct((ROWS, CHUNK), jnp.float32),
    ),
    scratch_types=(
        pltpu.VMEM((2, 4, 8, 128), jnp.float32),
        pltpu.VMEM((512, EMB), jnp.float32),
        pltpu.VMEM((4, 128), jnp.int32),
        pltpu.VMEM((1, 512), jnp.int32),
        pltpu.VMEM((16,), jnp.int32),
        pltpu.VMEM((CPW, CHUNK), jnp.int32),
        pltpu.VMEM((CPW, CHUNK), jnp.int32),
        pltpu.VMEM((CPW, CHUNK), jnp.int32),
        pltpu.VMEM((CPW, CHUNK), jnp.int32),
        pltpu.VMEM((CPW, CHUNK, L), jnp.float32),
        pltpu.VMEM((CPW, CHUNK, L), jnp.float32),
        pltpu.VMEM((CPW, CHUNK), jnp.float32),
        pltpu.VMEM((CPW, CHUNK), jnp.float32),
        pltpu.SemaphoreType.DMA,
    ),
    compiler_params=pltpu.CompilerParams(use_tc_tiling_on_sc=False,
                                         needs_layout_passes=False),
)
def _sc_k2(ucomp, upos, ucnt, mcomp, mpos, mcnt,
           uids2, mids2, ubias16, mbias16,
           urows_o, mrows_o, ub_o, mb_o,
           colb, rowb, posr, posf, cntv, uidx, midx, uhi, mhi,
           ubrows, mbrows, ubv, mbv, sem):
    wid = lax.axis_index("s") * NC + lax.axis_index("c")
    lanes = lax.iota(jnp.int32, L)

    def one_side(comp, pos, cnt_hbm, rows_o):
        pltpu.sync_copy(cnt_hbm.at[wid, pl.ds(0, 16)], cntv)
        cnt = cntv[pl.ds(0, L)][0]
        nit = (cnt + 511) >> 9

        zeros16 = jnp.zeros((L,), jnp.int32)

        def batch_iter(ci, _):
            pltpu.sync_copy(comp.at[:, pl.ds(wid * SLOTC + ci * 4, 4), :, :],
                            colb)
            pltpu.sync_copy(pos.at[wid, pl.ds(ci * 4, 4), :], posr)

            def sub(cj, _2):
                cjf = jnp.full((L,), 0, jnp.int32) + cj
                for g in range(8):
                    base16 = cj * 128 + g * L
                    idxg = ci * 512 + base16 + lanes
                    pvals = plsc.load_gather(posr, [cjf, g * L + lanes])
                    safe = jnp.where(idxg < cnt, pvals,
                                     jnp.full((L,), B, jnp.int32))
                    plsc.store_scatter(posf, [zeros16, base16 + lanes], safe)
                    for t in range(L):
                        row = plsc.load_gather(
                            colb, [lanes >> 3, cjf, lanes & 7,
                                   jnp.full((L,), g * L + t, jnp.int32)])
                        plsc.store_scatter(
                            rowb, [jnp.full((L,), 0, jnp.int32)
                                   + (base16 + t), lanes], row)
                return _2

            lax.fori_loop(0, 4, sub, _i32(0))
            pltpu.sync_copy(rowb, rows_o.at[posf.at[0]])
            return _

        lax.fori_loop(0, nit, batch_iter, _i32(0))

    one_side(ucomp, upos, ucnt, urows_o)
    one_side(mcomp, mpos, mcnt, mrows_o)

    r0 = wid * CPW
    pltpu.sync_copy(uids2.at[pl.ds(r0, CPW)], uidx)
    pltpu.sync_copy(mids2.at[pl.ds(r0, CPW)], midx)
    for j in range(CPW):
        for g in range(CHUNK // L):
            sl = pl.ds(g * L, L)
            uhi[j, sl] = uidx[j, sl] >> 4
            mhi[j, sl] = midx[j, sl] >> 4
    copies = []
    for j in range(CPW):
        copies.append(pltpu.async_copy(ubias16.at[uhi.at[j]], ubrows.at[j], sem))
        copies.append(pltpu.async_copy(mbias16.at[mhi.at[j]], mbrows.at[j], sem))
    for c in copies:
        c.wait()
    for j in range(CPW):
        jfull = jnp.full((L,), j, jnp.int32)
        for g in range(CHUNK // L):
            sl = pl.ds(g * L, L)
            rows = lanes + (g * L)
            ubv[j, sl] = plsc.load_gather(ubrows, [jfull, rows, uidx[j, sl] & 15])
            mbv[j, sl] = plsc.load_gather(mbrows, [jfull, rows, midx[j, sl] & 15])
    pltpu.sync_copy(ubv, ub_o.at[pl.ds(r0, CPW)])
    pltpu.sync_copy(mbv, mb_o.at[pl.ds(r0, CPW)])


def _tc_body(gb_ref, u_ref, m_ref, gp_ref, wb_ref, gbr_ref, ub_ref, mb_ref,
             out_ref):
    u = u_ref[...]
    m = m_ref[...]
    g = lax.dot_general(gp_ref[...], wb_ref[...], (((1,), (0,)), ((), ())),
                        preferred_element_type=jnp.float32)
    g = g + gbr_ref[...][None, :]
    r = u * m + (u + m) * g
    q = lax.broadcasted_iota(jnp.int32, (128, 8), 0) >> 4
    j = lax.broadcasted_iota(jnp.int32, (128, 8), 1)
    sel = (q == j).astype(jnp.float32)
    out8 = lax.dot_general(r, sel, (((1,), (0,)), ((), ())),
                           preferred_element_type=jnp.float32)
    out_ref[...] = out8 + ub_ref[...] + mb_ref[...] + gb_ref[0]


_tc_combine = pl.pallas_call(
    _tc_body,
    out_shape=jax.ShapeDtypeStruct((B // 8, 8), jnp.float32),
    in_specs=[pl.BlockSpec(memory_space=pltpu.SMEM)]
    + [pl.BlockSpec(memory_space=pltpu.VMEM)] * 7,
)


def kernel(user_ids, movie_ids, movie_genres, user_embedding, movie_embedding,
           global_bias, user_bias, movie_bias, genre_W, genre_b):
    uids = user_ids.astype(jnp.int32)
    mids = movie_ids.astype(jnp.int32)
    uembT = user_embedding.T
    membT = movie_embedding.T
    tailu = jnp.pad(uembT[:, TBASE_U:], ((0, 0), (0, 128 - (NU - TBASE_U))))
    tailm = jnp.pad(membT[:, TBASE_M:], ((0, 0), (0, 128 - (NM - TBASE_M))))
    ucomp, mcomp, upos, mpos, ucnt, mcnt = _sc_k1(
        uids, mids, uembT, membT, tailu, tailm)
    urp, mrp, ubv, mbv = _sc_k2(
        ucomp, upos, ucnt, mcomp, mpos, mcnt,
        uids.reshape(ROWS, CHUNK), mids.reshape(ROWS, CHUNK),
        user_bias.reshape(-1, L), movie_bias.reshape(-1, L))
    u_pack = urp[:B].reshape(B * EMB // 128, 128)
    m_pack = mrp[:B].reshape(B * EMB // 128, 128)
    genres_packed = movie_genres.astype(jnp.float32).reshape(B // 8, 8 * 26)
    wbig = jnp.kron(jnp.eye(8, dtype=jnp.float32), genre_W.T)
    gb_row = jnp.tile(genre_b, 8)
    pred = _tc_combine(global_bias, u_pack, m_pack, genres_packed, wbig,
                       gb_row, ubv.reshape(B // 8, 8), mbv.reshape(B // 8, 8))
    return pred.reshape(B)


# speedup vs baseline: 1.2584x; 1.2584x over previous
import functools

import jax
import jax.numpy as jnp
from jax import lax
from jax.experimental import pallas as pl
from jax.experimental.pallas import tpu as pltpu
from jax.experimental.pallas import tpu_sc as plsc

EMB = 16
B = 16384
NU = 1000000
NM = 100000
NC, NS = 2, 16
NW = NC * NS
L = 16
ROWS = B // 128
CHUNK = 128
CPW = 4

BLKW = 1152
TBASE_U = (NU // 128) * 128
TBASE_M = (NM // 128) * 128
RPW_U = 245 * 128
RPW_M = 25 * 128
NBLK_U = 28
NBLK_M = 4
OFFCAP_U = ((NU - BLKW) // 128) * 128
OFFCAP_M = ((NM - BLKW) // 128) * 128
CAP = B
SLOTC = CAP // 128

_mesh = plsc.VectorSubcoreMesh(core_axis_name="c", subcore_axis_name="s")


def _i32(x):
    return jnp.asarray(x, jnp.int32)


@functools.partial(
    pl.kernel,
    mesh=_mesh,
    out_type=(
        jax.ShapeDtypeStruct((2, NW * SLOTC, 8, 128), jnp.float32),
        jax.ShapeDtypeStruct((2, NW * SLOTC, 8, 128), jnp.float32),
        jax.ShapeDtypeStruct((NW, SLOTC, 128), jnp.int32),
        jax.ShapeDtypeStruct((NW, SLOTC, 128), jnp.int32),
        jax.ShapeDtypeStruct((NW, 128), jnp.int32),
        jax.ShapeDtypeStruct((NW, 128), jnp.int32),
    ),
    scratch_types=(
        pltpu.VMEM((B,), jnp.int32),
        pltpu.VMEM((B + 16,), jnp.int32),
        pltpu.VMEM((B + 16,), jnp.int32),
        pltpu.VMEM((48,), jnp.int32),
        pltpu.VMEM((48,), jnp.int32),
        pltpu.VMEM((L, BLKW + 128), jnp.float32),
        pltpu.VMEM((L, BLKW + 128), jnp.float32),
        pltpu.VMEM((2, 16, 8, 128), jnp.float32),
        pltpu.VMEM((2, 8, 128), jnp.int32),
        pltpu.VMEM((128,), jnp.int32),
        pltpu.SemaphoreType.DMA,
        pltpu.SemaphoreType.DMA,
    ),
    compiler_params=pltpu.CompilerParams(use_tc_tiling_on_sc=True,
                                         needs_layout_passes=False),
)
def _sc_k1(uids, mids, uembT, membT, tailu, tailm,
           ucomp, mcomp, upos_o, mpos_o, ucnt_o, mcnt_o,
           idsb, aid, apos, pend_i, pend_p, blkA, blkB, stage, pstage, cstage,
           semA, semB):
    wid = lax.axis_index("s") * NC + lax.axis_index("c")
    lanes = lax.iota(jnp.int32, L)

    def one_table(ids_hbm, tab, tail, tbase, rpw, nblk, offcap,
                  comp_o, pos_o, cnt_o):
        lo = wid * rpw
        hi = lo + rpw
        pltpu.sync_copy(ids_hbm, idsb)
        pltpu.sync_copy(tail, blkA.at[:, pl.ds(BLKW, 128)])
        pltpu.sync_copy(tail, blkB.at[:, pl.ds(BLKW, 128)])

        def scan_body(g, cnt):
            ids16 = idsb[pl.ds(g * L, L)]
            inr = (ids16 >= lo) & (ids16 < hi)
            c = plsc.cumsum(inr.astype(jnp.int32))
            offs = jnp.full((L,), cnt, jnp.int32) + c - 1
            pos16 = jnp.full((L,), g * L, jnp.int32) + lanes
            plsc.store_scatter(aid, [offs], ids16, mask=inr)
            plsc.store_scatter(apos, [offs], pos16, mask=inr)
            return cnt + c[L - 1]

        cnt = lax.fori_loop(0, B // L, scan_body, _i32(0), unroll=4)
        nvec = (cnt + (L - 1)) // L

        def extract_group(buf, base, k, koff, off2, kmask):
            t16p = pend_p[pl.ds(base, L)]
            t16i = pend_i[pl.ds(base, L)]
            lcol = jnp.where(t16i >= tbase,
                             t16i - (tbase - BLKW), t16i - off2)
            lcol = jnp.clip(lcol, 0, BLKW + 127)
            slots = jnp.full((L,), koff, jnp.int32) + lanes
            smask = (lanes < k) if kmask else None
            plsc.store_scatter(
                pstage,
                [(slots >> 10) & 1, (slots >> 7) & 7, slots & 127],
                t16p, mask=smask)
            for t in range(L):
                cs = koff + t
                col = plsc.load_gather(
                    buf, [lanes, jnp.full((L,), lcol[t], jnp.int32)])
                m = None
                if kmask:
                    m = (jnp.full((L,), t, jnp.int32)
                         < jnp.full((L,), k, jnp.int32))
                plsc.store_scatter(
                    stage,
                    [lanes >> 3, jnp.full((L,), (cs >> 7) & 15, jnp.int32),
                     lanes & 7, jnp.full((L,), cs & 127, jnp.int32)],
                    col, mask=m)

        def flush(f):
            c = f & 1
            pltpu.sync_copy(
                stage.at[:, pl.ds(c * 8, 8)],
                comp_o.at[:, pl.ds(wid * SLOTC + f * 8, 8)])
            pltpu.sync_copy(pstage.at[c], pos_o.at[wid, pl.ds(f * 8, 8), :])

        def append(dst, x, m, base):
            c = plsc.cumsum(m.astype(jnp.int32))
            offs = jnp.full((L,), base, jnp.int32) + c - 1
            plsc.store_scatter(dst, [offs], x, mask=m)

        def start(bi, buf, sem):
            off = lo + bi * BLKW
            off2 = jnp.minimum(off, offcap)
            pltpu.async_copy(tab.at[:, pl.ds(off2, BLKW)],
                             buf.at[:, pl.ds(0, BLKW)], sem)

        def drain(buf, sem):
            pltpu.make_async_copy(tab.at[:, pl.ds(0, BLKW)],
                                  buf.at[:, pl.ds(0, BLKW)], sem).wait()

        def process(buf, bi, s):
            off = lo + bi * BLKW
            off2 = jnp.minimum(off, offcap)

            def scan_blk(v, carry):
                s1, p = carry
                pos16 = apos[pl.ds(v * L, L)]
                ids16 = aid[pl.ds(v * L, L)]
                valid = (jnp.full((L,), v * L, jnp.int32) + lanes) < cnt
                inb = (ids16 >= off) & (ids16 < off + BLKW) & valid
                kk = plsc.all_reduce_population_count(inb)[0]

                @pl.when(kk > 0)
                def _():
                    append(pend_i, ids16, inb, p)
                    append(pend_p, pos16, inb, p)

                p2 = p + kk

                @pl.when(p2 >= L)
                def _():
                    extract_group(buf, p2 - L, L, s1, off2, kmask=False)

                consumed = jnp.where(p2 >= L, L, 0)
                s2 = s1 + consumed

                @pl.when((s2 >> 10) != (s1 >> 10))
                def _():
                    flush(s1 >> 10)

                return (s2, p2 - consumed)

            s_a, p_a = lax.fori_loop(0, nvec, scan_blk, (s, _i32(0)))

            @pl.when(p_a > 0)
            def _():
                extract_group(buf, 0, p_a, s_a, off2, kmask=True)

            s_b = s_a + p_a

            @pl.when((s_b >> 10) != (s_a >> 10))
            def _():
                flush(s_a >> 10)

            return s_b

        start(_i32(0), blkA, semA)
        start(_i32(1), blkB, semB)

        def pair_iter(bp, s):
            bi0 = bp * 2
            drain(blkA, semA)
            s = process(blkA, bi0, s)
            start(bi0 + 2, blkA, semA)
            drain(blkB, semB)
            s = process(blkB, bi0 + 1, s)
            start(bi0 + 3, blkB, semB)
            return s

        s_fin = lax.fori_loop(0, nblk // 2, pair_iter, _i32(0))
        drain(blkA, semA)
        drain(blkB, semB)

        @pl.when((s_fin & 1023) != 0)
        def _():
            flush(s_fin >> 10)

        for g in range(8):
            cstage[pl.ds(g * L, L)] = jnp.full((L,), cnt, jnp.int32)
        pltpu.sync_copy(cstage, cnt_o.at[wid])

    one_table(uids, uembT, tailu, TBASE_U, RPW_U, NBLK_U, OFFCAP_U,
              ucomp, upos_o, ucnt_o)
    one_table(mids, membT, tailm, TBASE_M, RPW_M, NBLK_M, OFFCAP_M,
              mcomp, mpos_o, mcnt_o)


@functools.partial(
    pl.kernel,
    mesh=_mesh,
    out_type=(
        jax.ShapeDtypeStruct((B + 128, EMB), jnp.float32),
        jax.ShapeDtypeStruct((B + 128, EMB), jnp.float32),
        jax.ShapeDtypeStruct((ROWS, CHUNK), jnp.float32),
        jax.ShapeDtypeStruct((ROWS, CHUNK), jnp.float32),
    ),
    scratch_types=(
        pltpu.VMEM((2, 8, 128), jnp.float32),
        pltpu.VMEM((128, EMB), jnp.float32),
        pltpu.VMEM((1, 128), jnp.int32),
        pltpu.VMEM((1, 128), jnp.int32),
        pltpu.VMEM((16,), jnp.int32),
        pltpu.VMEM((CPW, CHUNK), jnp.int32),
        pltpu.VMEM((CPW, CHUNK), jnp.int32),
        pltpu.VMEM((CPW, CHUNK), jnp.int32),
        pltpu.VMEM((CPW, CHUNK), jnp.int32),
        pltpu.VMEM((CPW, CHUNK, L), jnp.float32),
        pltpu.VMEM((CPW, CHUNK, L), jnp.float32),
        pltpu.VMEM((CPW, CHUNK), jnp.float32),
        pltpu.VMEM((CPW, CHUNK), jnp.float32),
        pltpu.SemaphoreType.DMA,
    ),
    compiler_params=pltpu.CompilerParams(use_tc_tiling_on_sc=False,
                                         needs_layout_passes=False),
)
def _sc_k2(ucomp, upos, ucnt, mcomp, mpos, mcnt,
           uids2, mids2, ubias16, mbias16,
           urows_o, mrows_o, ub_o, mb_o,
           colb, rowb, posr, posf, cntv, uidx, midx, uhi, mhi,
           ubrows, mbrows, ubv, mbv, sem):
    wid = lax.axis_index("s") * NC + lax.axis_index("c")
    lanes = lax.iota(jnp.int32, L)

    def one_side(comp, pos, cnt_hbm, rows_o):
        pltpu.sync_copy(cnt_hbm.at[wid, pl.ds(0, 16)], cntv)
        cnt = cntv[pl.ds(0, L)][0]
        nit = (cnt + 127) >> 7

        def chunk_iter(ch, _):
            pltpu.sync_copy(comp.at[:, wid * SLOTC + ch, :, :], colb)
            pltpu.sync_copy(pos.at[wid, ch], posr.at[0])
            for g in range(8):
                base = ch * 128 + g * L
                idx16 = jnp.full((L,), 0, jnp.int32) + base + lanes
                safe = jnp.where(idx16 < cnt, posr[0, pl.ds(g * L, L)],
                                 jnp.full((L,), B, jnp.int32))
                posf[0, pl.ds(g * L, L)] = safe
                for t in range(L):
                    row = plsc.load_gather(
                        colb, [lanes >> 3, lanes & 7,
                               jnp.full((L,), g * L + t, jnp.int32)])
                    rowb[g * L + t, pl.ds(0, L)] = row
            pltpu.sync_copy(rowb, rows_o.at[posf.at[0]])
            return _

        lax.fori_loop(0, nit, chunk_iter, _i32(0))

    one_side(ucomp, upos, ucnt, urows_o)
    one_side(mcomp, mpos, mcnt, mrows_o)

    r0 = wid * CPW
    pltpu.sync_copy(uids2.at[pl.ds(r0, CPW)], uidx)
    pltpu.sync_copy(mids2.at[pl.ds(r0, CPW)], midx)
    for j in range(CPW):
        for g in range(CHUNK // L):
            sl = pl.ds(g * L, L)
            uhi[j, sl] = uidx[j, sl] >> 4
            mhi[j, sl] = midx[j, sl] >> 4
    copies = []
    for j in range(CPW):
        copies.append(pltpu.async_copy(ubias16.at[uhi.at[j]], ubrows.at[j], sem))
        copies.append(pltpu.async_copy(mbias16.at[mhi.at[j]], mbrows.at[j], sem))
    for c in copies:
        c.wait()
    for j in range(CPW):
        jfull = jnp.full((L,), j, jnp.int32)
        for g in range(CHUNK // L):
            sl = pl.ds(g * L, L)
            rows = lanes + (g * L)
            ubv[j, sl] = plsc.load_gather(ubrows, [jfull, rows, uidx[j, sl] & 15])
            mbv[j, sl] = plsc.load_gather(mbrows, [jfull, rows, midx[j, sl] & 15])
    pltpu.sync_copy(ubv, ub_o.at[pl.ds(r0, CPW)])
    pltpu.sync_copy(mbv, mb_o.at[pl.ds(r0, CPW)])


def _tc_body(gb_ref, u_ref, m_ref, gp_ref, wb_ref, gbr_ref, ub_ref, mb_ref,
             out_ref):
    u = u_ref[...]
    m = m_ref[...]
    g = lax.dot_general(gp_ref[...], wb_ref[...], (((1,), (0,)), ((), ())),
                        preferred_element_type=jnp.float32)
    g = g + gbr_ref[...][None, :]
    r = u * m + (u + m) * g
    q = lax.broadcasted_iota(jnp.int32, (128, 8), 0) >> 4
    j = lax.broadcasted_iota(jnp.int32, (128, 8), 1)
    sel = (q == j).astype(jnp.float32)
    out8 = lax.dot_general(r, sel, (((1,), (0,)), ((), ())),
                           preferred_element_type=jnp.float32)
    out_ref[...] = out8 + ub_ref[...] + mb_ref[...] + gb_ref[0]


_tc_combine = pl.pallas_call(
    _tc_body,
    out_shape=jax.ShapeDtypeStruct((B // 8, 8), jnp.float32),
    in_specs=[pl.BlockSpec(memory_space=pltpu.SMEM)]
    + [pl.BlockSpec(memory_space=pltpu.VMEM)] * 7,
)


def kernel(user_ids, movie_ids, movie_genres, user_embedding, movie_embedding,
           global_bias, user_bias, movie_bias, genre_W, genre_b):
    uids = user_ids.astype(jnp.int32)
    mids = movie_ids.astype(jnp.int32)
    uembT = user_embedding.T
    membT = movie_embedding.T
    tailu = jnp.pad(uembT[:, TBASE_U:], ((0, 0), (0, 128 - (NU - TBASE_U))))
    tailm = jnp.pad(membT[:, TBASE_M:], ((0, 0), (0, 128 - (NM - TBASE_M))))
    ucomp, mcomp, upos, mpos, ucnt, mcnt = _sc_k1(
        uids, mids, uembT, membT, tailu, tailm)
    urp, mrp, ubv, mbv = _sc_k2(
        ucomp, upos, ucnt, mcomp, mpos, mcnt,
        uids.reshape(ROWS, CHUNK), mids.reshape(ROWS, CHUNK),
        user_bias.reshape(-1, L), movie_bias.reshape(-1, L))
    u_pack = urp[:B].reshape(B * EMB // 128, 128)
    m_pack = mrp[:B].reshape(B * EMB // 128, 128)
    genres_packed = movie_genres.astype(jnp.float32).reshape(B // 8, 8 * 26)
    wbig = jnp.kron(jnp.eye(8, dtype=jnp.float32), genre_W.T)
    gb_row = jnp.tile(genre_b, 8)
    pred = _tc_combine(global_bias, u_pack, m_pack, genres_packed, wbig,
                       gb_row, ubv.reshape(B // 8, 8), mbv.reshape(B // 8, 8))
    return pred.reshape(B)
